# baseline (speedup 1.0000x reference)
import functools

import jax
import jax.numpy as jnp
from jax import lax
from jax.experimental import pallas as pl
from jax.experimental.pallas import tpu as pltpu
from jax.experimental.pallas import tpu_sc as plsc

_NC, _NS, _L = 2, 16, 16
_NW = _NC * _NS
_CHUNK = 128


def _stats_body(p_ref, q_ref):
    p = p_ref[...]
    q_ref[...] = lax.dot_general(
        p, p, (((0,), (0,)), ((), ())), preferred_element_type=jnp.float32)


def _make_main_body(share):
    def _main_body(x_ref, p_ref, w_ref, wx_ref, a_ref, c_ref, b2_ref,
                   bias_ref, bx_ref, sv_ref, gmax_ref):
        x = x_ref[...]
        s = jnp.dot(x, w_ref[...], preferred_element_type=jnp.float32)
        hr = jnp.maximum(
            jnp.dot(p_ref[...], a_ref[...],
                    preferred_element_type=jnp.float32) + c_ref[...], 0.0)
        s = s + jnp.dot(hr, b2_ref[...],
                        preferred_element_type=jnp.float32) + bias_ref[...]
        v = jnp.dot(x, wx_ref[...], preferred_element_type=jnp.float32) + bx_ref[...]
        sv_ref[...] = jnp.concatenate([s, v], axis=1)

        @pl.when(pl.program_id(0) == 0)
        def _():
            gmax_ref[0, 0] = -jnp.inf

        gmax_ref[0, 0] = jnp.maximum(gmax_ref[0, 0], jnp.max(s))
    return _main_body


def _exp_body(sv_ref, gmax_ref, t_ref):
    sv = sv_ref[...]
    hid = sv.shape[1] // 2
    e = jnp.exp(sv[:, :hid] - gmax_ref[0, 0])
    t_ref[...] = jnp.concatenate(
        [e, sv[:, hid:] * e, jnp.zeros_like(sv)], axis=1)


def _make_scatter(m, n_pad):
    nchunks = m // _CHUNK
    iters = (nchunks + _NW - 1) // _NW
    rz = n_pad // _NS
    mesh = plsc.VectorSubcoreMesh(core_axis_name="c", subcore_axis_name="s")

    @functools.partial(
        pl.kernel,
        out_type=jax.ShapeDtypeStruct((_NC * n_pad, 8 * _L), jnp.float32),
        mesh=mesh,
        scratch_types=[
            pltpu.VMEM((_CHUNK,), jnp.int32),
            pltpu.VMEM((_CHUNK, 8 * _L), jnp.float32),
            pltpu.VMEM((8, 8 * _L), jnp.float32),
            pltpu.VMEM_SHARED((n_pad, 8 * _L), jnp.float32),
        ],
    )
    def _scatter(sv_hbm, idx_hbm, out_hbm, idxb, svb, zb, acc):
        cid = lax.axis_index("c")
        sid = lax.axis_index("s")
        wid = sid * _NC + cid

        def zrow(r, carry):
            for j in range(8):
                zb[r, pl.ds(j * _L, _L)] = jnp.zeros((_L,), jnp.float32)
            return carry

        lax.fori_loop(0, 8, zrow, 0)

        def zslab(r, carry):
            pltpu.sync_copy(zb, acc.at[pl.ds(sid * rz + r * 8, 8)])
            return carry

        lax.fori_loop(0, rz // 8, zslab, 0)
        plsc.subcore_barrier()

        def body(j, carry):
            chunk = wid + _NW * j

            @pl.when(chunk < nchunks)
            def _():
                base = chunk * _CHUNK
                pltpu.sync_copy(idx_hbm.at[pl.ds(base, _CHUNK)], idxb)
                pltpu.sync_copy(sv_hbm.at[pl.ds(base, _CHUNK)], svb)
                pltpu.sync_copy(svb, acc.at[idxb], add=True)

            return carry

        lax.fori_loop(0, iters, body, 0)
        plsc.subcore_barrier()
        pltpu.sync_copy(acc.at[pl.ds(sid * rz, rz)],
                        out_hbm.at[pl.ds(cid * n_pad + sid * rz, rz)])

    return _scatter


def _make_combine_body(hid, share):
    def _combine_body(p0_ref, p1_ref, x_ref, out_ref):
        den = p0_ref[:, 0:hid] + p1_ref[:, 0:hid]
        num = p0_ref[:, hid:2 * hid] + p1_ref[:, hid:2 * hid]
        res = jnp.where(den > 0.0, num / den, 0.0)
        out_ref[...] = x_ref[...] + jnp.concatenate([res] * share, axis=1)
    return _combine_body


def kernel(x, x_knn, knn_idx, p_r, W, b, Wx, bx, Wp1, gamma, beta, Wp2, bp2):
    n, k, c = x_knn.shape
    hid = W.shape[1]
    share = c // hid
    m = n * k
    f32 = jnp.float32

    xe = x_knn.reshape(m, c)
    p49 = jnp.concatenate(
        [p_r.reshape(n, k * 3), jnp.ones((n, 1), f32)], axis=1)

    q = pl.pallas_call(
        _stats_body,
        out_shape=jax.ShapeDtypeStruct((k * 3 + 1, k * 3 + 1), f32),
    )(p49)

    q48 = q[:k * 3, :k * 3].reshape(k, 3, k, 3)
    c3 = jnp.einsum('iaib->ab', q48)
    s3 = q[k * 3, :k * 3].reshape(k, 3).sum(axis=0)
    mean = (s3 / m) @ Wp1
    eh2 = jnp.einsum('ij,ik,kj->j', Wp1, c3 / m, Wp1)
    var = eh2 - mean * mean
    a = gamma * lax.rsqrt(var + 1e-5)
    cshift = beta - mean * a
    a4 = jnp.zeros((4, 4), f32).at[:3, :3].set(Wp1 * a[None, :])
    c4 = jnp.zeros((1, 4), f32).at[0, :3].set(cshift)
    b24 = jnp.zeros((4, hid), f32).at[:3, :].set(Wp2 @ W)
    b2 = (b + bp2 @ W).reshape(1, hid)
    bx2 = bx.reshape(1, hid)
    p4 = jnp.pad(p_r.reshape(m, 3), ((0, 0), (0, 1)))

    be = 2000
    sv, gmax = pl.pallas_call(
        _make_main_body(share),
        grid=(m // be,),
        in_specs=[
            pl.BlockSpec((be, c), lambda i: (i, 0)),
            pl.BlockSpec((be, 4), lambda i: (i, 0)),
            pl.BlockSpec((c, hid), lambda i: (0, 0)),
            pl.BlockSpec((c, hid), lambda i: (0, 0)),
            pl.BlockSpec((4, 4), lambda i: (0, 0)),
            pl.BlockSpec((1, 4), lambda i: (0, 0)),
            pl.BlockSpec((4, hid), lambda i: (0, 0)),
            pl.BlockSpec((1, hid), lambda i: (0, 0)),
            pl.BlockSpec((1, hid), lambda i: (0, 0)),
        ],
        out_specs=[
            pl.BlockSpec((be, 2 * hid), lambda i: (i, 0)),
            pl.BlockSpec((1, 1), lambda i: (0, 0), memory_space=pltpu.SMEM),
        ],
        out_shape=[
            jax.ShapeDtypeStruct((m, 2 * hid), f32),
            jax.ShapeDtypeStruct((1, 1), f32),
        ],
    )(xe, p4, W, Wx, a4, c4, b24, b2, bx2)

    t = pl.pallas_call(
        _exp_body,
        grid=(m // be,),
        in_specs=[
            pl.BlockSpec((be, 2 * hid), lambda i: (i, 0)),
            pl.BlockSpec((1, 1), lambda i: (0, 0), memory_space=pltpu.SMEM),
        ],
        out_specs=pl.BlockSpec((be, 8 * _L), lambda i: (i, 0)),
        out_shape=jax.ShapeDtypeStruct((m, 8 * _L), f32),
    )(sv, gmax)

    n_pad = ((n + _NS * 8 - 1) // (_NS * 8)) * (_NS * 8)
    parts = _make_scatter(m, n_pad)(t, knn_idx.reshape(m))
    p0 = lax.slice(parts, (0, 0), (n, 2 * hid))
    p1 = lax.slice(parts, (n_pad, 0), (n_pad + n, 2 * hid))

    bn = 2000
    nb = n // bn
    out = pl.pallas_call(
        _make_combine_body(hid, share),
        grid=(nb,),
        in_specs=[
            pl.BlockSpec((bn, 2 * hid), lambda i: (i, 0)),
            pl.BlockSpec((bn, 2 * hid), lambda i: (i, 0)),
            pl.BlockSpec((bn, c), lambda i: (i, 0)),
        ],
        out_specs=pl.BlockSpec((bn, c), lambda i: (i, 0)),
        out_shape=jax.ShapeDtypeStruct((n, c), f32),
    )(p0, p1, x)
    return out


# speedup vs baseline: 1.2114x; 1.2114x over previous
import functools

import jax
import jax.numpy as jnp
from jax import lax
from jax.experimental import pallas as pl
from jax.experimental.pallas import tpu as pltpu
from jax.experimental.pallas import tpu_sc as plsc

_NC, _NS, _L = 2, 16, 16
_NW = _NC * _NS
_CHUNK = 128


def _stats_body(p_ref, q_ref):
    p = p_ref[...]
    q_ref[...] = lax.dot_general(
        p, p, (((0,), (0,)), ((), ())), preferred_element_type=jnp.float32)


def _main_body(xa_ref, xb_ref, pa_ref, pb_ref, w_ref, wx_ref, a_ref, c_ref,
               b2_ref, bias_ref, bx_ref, tp_ref, gmax_ref):
    def half(x, p):
        s = jnp.dot(x, w_ref[...], preferred_element_type=jnp.float32)
        hr = jnp.maximum(
            jnp.dot(p, a_ref[...],
                    preferred_element_type=jnp.float32) + c_ref[...], 0.0)
        s = s + jnp.dot(hr, b2_ref[...],
                        preferred_element_type=jnp.float32) + bias_ref[...]
        v = jnp.dot(x, wx_ref[...], preferred_element_type=jnp.float32) + bx_ref[...]
        return s, v

    sa, va = half(xa_ref[...], pa_ref[...])
    sb, vb = half(xb_ref[...], pb_ref[...])
    tp_ref[...] = jnp.concatenate([sa, va, sb, vb], axis=1)

    @pl.when(pl.program_id(0) == 0)
    def _():
        gmax_ref[0, 0] = -jnp.inf

    gmax_ref[0, 0] = jnp.maximum(
        gmax_ref[0, 0], jnp.maximum(jnp.max(sa), jnp.max(sb)))


def _make_scatter(mh, n_pad):
    nchunks = mh // _CHUNK
    iters = (nchunks + _NW - 1) // _NW
    rz = n_pad // _NS
    mesh = plsc.VectorSubcoreMesh(core_axis_name="c", subcore_axis_name="s")

    @functools.partial(
        pl.kernel,
        out_type=jax.ShapeDtypeStruct((_NC * n_pad, 8 * _L), jnp.float32),
        mesh=mesh,
        scratch_types=[
            pltpu.VMEM((_CHUNK,), jnp.int32),
            pltpu.VMEM((_CHUNK,), jnp.int32),
            pltpu.VMEM((_CHUNK, 8 * _L), jnp.float32),
            pltpu.VMEM((_CHUNK, 8 * _L), jnp.float32),
            pltpu.VMEM((_CHUNK, 8 * _L), jnp.float32),
            pltpu.VMEM((_L,), jnp.float32),
            pltpu.VMEM_SHARED((n_pad, 8 * _L), jnp.float32),
        ],
    )
    def _scatter(tp_hbm, ia_hbm, ib_hbm, g_hbm, out_hbm,
                 iab, ibb, svb, sva, svb2, gb, acc):
        cid = lax.axis_index("c")
        sid = lax.axis_index("s")
        wid = sid * _NC + cid

        def zpay(r, carry):
            for j in range(8):
                svb[r, pl.ds(j * _L, _L)] = jnp.zeros((_L,), jnp.float32)
            for j in range(4, 8):
                sva[r, pl.ds(j * _L, _L)] = jnp.zeros((_L,), jnp.float32)
                svb2[r, pl.ds(j * _L, _L)] = jnp.zeros((_L,), jnp.float32)
            return carry

        lax.fori_loop(0, _CHUNK, zpay, 0)

        def zslab(r, carry):
            pltpu.sync_copy(svb.at[pl.ds(0, 8)],
                            acc.at[pl.ds(sid * rz + r * 8, 8)])
            return carry

        lax.fori_loop(0, rz // 8, zslab, 0)
        pltpu.sync_copy(g_hbm, gb)
        gv = gb[...]
        plsc.subcore_barrier()

        def body(j, carry):
            chunk = wid + _NW * j

            @pl.when(chunk < nchunks)
            def _():
                base = chunk * _CHUNK
                pltpu.sync_copy(ia_hbm.at[pl.ds(base, _CHUNK)], iab)
                pltpu.sync_copy(ib_hbm.at[pl.ds(base, _CHUNK)], ibb)
                pltpu.sync_copy(tp_hbm.at[pl.ds(base, _CHUNK)], svb)

                def rbody(r, c2):
                    ea0 = jnp.exp(svb[r, pl.ds(0, _L)] - gv)
                    ea1 = jnp.exp(svb[r, pl.ds(_L, _L)] - gv)
                    eb0 = jnp.exp(svb[r, pl.ds(4 * _L, _L)] - gv)
                    eb1 = jnp.exp(svb[r, pl.ds(5 * _L, _L)] - gv)
                    sva[r, pl.ds(0, _L)] = ea0
                    sva[r, pl.ds(_L, _L)] = ea1
                    sva[r, pl.ds(2 * _L, _L)] = svb[r, pl.ds(2 * _L, _L)] * ea0
                    sva[r, pl.ds(3 * _L, _L)] = svb[r, pl.ds(3 * _L, _L)] * ea1
                    svb2[r, pl.ds(0, _L)] = eb0
                    svb2[r, pl.ds(_L, _L)] = eb1
                    svb2[r, pl.ds(2 * _L, _L)] = svb[r, pl.ds(6 * _L, _L)] * eb0
                    svb2[r, pl.ds(3 * _L, _L)] = svb[r, pl.ds(7 * _L, _L)] * eb1
                    return c2

                lax.fori_loop(0, _CHUNK, rbody, 0)
                pltpu.sync_copy(sva, acc.at[iab], add=True)
                pltpu.sync_copy(svb2, acc.at[ibb], add=True)

            return carry

        lax.fori_loop(0, iters, body, 0)
        plsc.subcore_barrier()
        pltpu.sync_copy(acc.at[pl.ds(sid * rz, rz)],
                        out_hbm.at[pl.ds(cid * n_pad + sid * rz, rz)])

    return _scatter


def _make_combine_body(hid, share):
    def _combine_body(p0_ref, p1_ref, x_ref, out_ref):
        den = p0_ref[:, 0:hid] + p1_ref[:, 0:hid]
        num = p0_ref[:, hid:2 * hid] + p1_ref[:, hid:2 * hid]
        res = jnp.where(den > 0.0, num / den, 0.0)
        out_ref[...] = x_ref[...] + jnp.concatenate([res] * share, axis=1)
    return _combine_body


def kernel(x, x_knn, knn_idx, p_r, W, b, Wx, bx, Wp1, gamma, beta, Wp2, bp2):
    n, k, c = x_knn.shape
    hid = W.shape[1]
    share = c // hid
    m = n * k
    mh = m // 2
    f32 = jnp.float32

    xe = x_knn.reshape(m, c)
    p49 = jnp.concatenate(
        [p_r.reshape(n, k * 3), jnp.ones((n, 1), f32)], axis=1)

    q = pl.pallas_call(
        _stats_body,
        out_shape=jax.ShapeDtypeStruct((k * 3 + 1, k * 3 + 1), f32),
    )(p49)

    q48 = q[:k * 3, :k * 3].reshape(k, 3, k, 3)
    c3 = jnp.einsum('iaib->ab', q48)
    s3 = q[k * 3, :k * 3].reshape(k, 3).sum(axis=0)
    mean = (s3 / m) @ Wp1
    eh2 = jnp.einsum('ij,ik,kj->j', Wp1, c3 / m, Wp1)
    var = eh2 - mean * mean
    a = gamma * lax.rsqrt(var + 1e-5)
    cshift = beta - mean * a
    a4 = jnp.zeros((4, 4), f32).at[:3, :3].set(Wp1 * a[None, :])
    c4 = jnp.zeros((1, 4), f32).at[0, :3].set(cshift)
    b24 = jnp.zeros((4, hid), f32).at[:3, :].set(Wp2 @ W)
    b2 = (b + bp2 @ W).reshape(1, hid)
    bx2 = bx.reshape(1, hid)
    p4 = jnp.pad(p_r.reshape(m, 3), ((0, 0), (0, 1)))

    be = 1000
    nbh = mh // be
    tp, gmax = pl.pallas_call(
        _main_body,
        grid=(nbh,),
        in_specs=[
            pl.BlockSpec((be, c), lambda i: (i, 0)),
            pl.BlockSpec((be, c), lambda i: (i + nbh, 0)),
            pl.BlockSpec((be, 4), lambda i: (i, 0)),
            pl.BlockSpec((be, 4), lambda i: (i + nbh, 0)),
            pl.BlockSpec((c, hid), lambda i: (0, 0)),
            pl.BlockSpec((c, hid), lambda i: (0, 0)),
            pl.BlockSpec((4, 4), lambda i: (0, 0)),
            pl.BlockSpec((1, 4), lambda i: (0, 0)),
            pl.BlockSpec((4, hid), lambda i: (0, 0)),
            pl.BlockSpec((1, hid), lambda i: (0, 0)),
            pl.BlockSpec((1, hid), lambda i: (0, 0)),
        ],
        out_specs=[
            pl.BlockSpec((be, 8 * _L), lambda i: (i, 0)),
            pl.BlockSpec((1, 1), lambda i: (0, 0), memory_space=pltpu.SMEM),
        ],
        out_shape=[
            jax.ShapeDtypeStruct((mh, 8 * _L), f32),
            jax.ShapeDtypeStruct((1, 1), f32),
        ],
    )(xe, xe, p4, p4, W, Wx, a4, c4, b24, b2, bx2)

    gvec = jnp.full((_L,), gmax[0, 0], f32)

    n_pad = ((n + _NS * 8 - 1) // (_NS * 8)) * (_NS * 8)
    idx = knn_idx.reshape(m)
    parts = _make_scatter(mh, n_pad)(tp, idx[:mh], idx[mh:], gvec)
    pc0 = lax.slice(parts, (0, 0), (n, 2 * hid))
    pc1 = lax.slice(parts, (n_pad, 0), (n_pad + n, 2 * hid))

    bn = 2000
    nb = n // bn
    out = pl.pallas_call(
        _make_combine_body(hid, share),
        grid=(nb,),
        in_specs=[
            pl.BlockSpec((bn, 2 * hid), lambda i: (i, 0)),
            pl.BlockSpec((bn, 2 * hid), lambda i: (i, 0)),
            pl.BlockSpec((bn, c), lambda i: (i, 0)),
        ],
        out_specs=pl.BlockSpec((bn, c), lambda i: (i, 0)),
        out_shape=jax.ShapeDtypeStruct((n, c), f32),
    )(pc0, pc1, x)
    return out


# speedup vs baseline: 1.2464x; 1.0288x over previous
import functools

import jax
import jax.numpy as jnp
from jax import lax
from jax.experimental import pallas as pl
from jax.experimental.pallas import tpu as pltpu
from jax.experimental.pallas import tpu_sc as plsc

_NC, _NS, _L = 2, 16, 16
_NW = _NC * _NS
_CHUNK = 128


def _stats_body(p_ref, q_ref):
    p = p_ref[...]
    q_ref[...] = lax.dot_general(
        p, p, (((0,), (0,)), ((), ())), preferred_element_type=jnp.float32)


def _main_body(xa_ref, xb_ref, pa_ref, pb_ref, w_ref, wx_ref, a_ref, c_ref,
               b2_ref, bias_ref, bx_ref, tp_ref, gmax_ref):
    def half(x, p):
        s = jnp.dot(x, w_ref[...], preferred_element_type=jnp.float32)
        hr = jnp.maximum(
            jnp.dot(p, a_ref[...],
                    preferred_element_type=jnp.float32) + c_ref[...], 0.0)
        s = s + jnp.dot(hr, b2_ref[...],
                        preferred_element_type=jnp.float32) + bias_ref[...]
        v = jnp.dot(x, wx_ref[...], preferred_element_type=jnp.float32) + bx_ref[...]
        return s, v

    sa, va = half(xa_ref[...], pa_ref[...])
    sb, vb = half(xb_ref[...], pb_ref[...])
    tp_ref[...] = jnp.concatenate([sa, va, sb, vb], axis=1)

    @pl.when(pl.program_id(0) == 0)
    def _():
        gmax_ref[0, 0] = -jnp.inf

    gmax_ref[0, 0] = jnp.maximum(
        gmax_ref[0, 0], jnp.maximum(jnp.max(sa), jnp.max(sb)))


def _make_scatter(mh, n_pad):
    nchunks = mh // _CHUNK
    iters = (nchunks + _NW - 1) // _NW
    rz = n_pad // _NS
    mesh = plsc.VectorSubcoreMesh(core_axis_name="c", subcore_axis_name="s")

    @functools.partial(
        pl.kernel,
        out_type=jax.ShapeDtypeStruct((_NC * n_pad, 8 * _L), jnp.float32),
        mesh=mesh,
        scratch_types=[
            pltpu.VMEM((_CHUNK,), jnp.int32),
            pltpu.VMEM((_CHUNK,), jnp.int32),
            pltpu.VMEM((_CHUNK, 8 * _L), jnp.float32),
            pltpu.VMEM((_CHUNK, 8 * _L), jnp.float32),
            pltpu.VMEM((_CHUNK, 8 * _L), jnp.float32),
            pltpu.VMEM((_L,), jnp.float32),
            pltpu.VMEM_SHARED((n_pad, 8 * _L), jnp.float32),
        ],
    )
    def _scatter(tp_hbm, ia_hbm, ib_hbm, g_hbm, out_hbm,
                 iab, ibb, svb, sva, svb2, gb, acc):
        cid = lax.axis_index("c")
        sid = lax.axis_index("s")
        wid = sid * _NC + cid

        def zpay(r, carry):
            for j in range(8):
                svb[r, pl.ds(j * _L, _L)] = jnp.zeros((_L,), jnp.float32)
            for j in range(4, 8):
                sva[r, pl.ds(j * _L, _L)] = jnp.zeros((_L,), jnp.float32)
                svb2[r, pl.ds(j * _L, _L)] = jnp.zeros((_L,), jnp.float32)
            return carry

        lax.fori_loop(0, _CHUNK, zpay, 0)

        def zslab(r, carry):
            pltpu.sync_copy(svb.at[pl.ds(0, 8)],
                            acc.at[pl.ds(sid * rz + r * 8, 8)])
            return carry

        lax.fori_loop(0, rz // 8, zslab, 0)
        pltpu.sync_copy(g_hbm, gb)
        gv = gb[...]
        plsc.subcore_barrier()

        def body(j, carry):
            chunk = wid + _NW * j

            @pl.when(chunk < nchunks)
            def _():
                base = chunk * _CHUNK
                pltpu.sync_copy(ia_hbm.at[pl.ds(base, _CHUNK)], iab)
                pltpu.sync_copy(ib_hbm.at[pl.ds(base, _CHUNK)], ibb)
                pltpu.sync_copy(tp_hbm.at[pl.ds(base, _CHUNK)], svb)

                def rbody(r, c2):
                    ea0 = jnp.exp(svb[r, pl.ds(0, _L)] - gv)
                    ea1 = jnp.exp(svb[r, pl.ds(_L, _L)] - gv)
                    eb0 = jnp.exp(svb[r, pl.ds(4 * _L, _L)] - gv)
                    eb1 = jnp.exp(svb[r, pl.ds(5 * _L, _L)] - gv)
                    sva[r, pl.ds(0, _L)] = ea0
                    sva[r, pl.ds(_L, _L)] = ea1
                    sva[r, pl.ds(2 * _L, _L)] = svb[r, pl.ds(2 * _L, _L)] * ea0
                    sva[r, pl.ds(3 * _L, _L)] = svb[r, pl.ds(3 * _L, _L)] * ea1
                    svb2[r, pl.ds(0, _L)] = eb0
                    svb2[r, pl.ds(_L, _L)] = eb1
                    svb2[r, pl.ds(2 * _L, _L)] = svb[r, pl.ds(6 * _L, _L)] * eb0
                    svb2[r, pl.ds(3 * _L, _L)] = svb[r, pl.ds(7 * _L, _L)] * eb1
                    return c2

                lax.fori_loop(0, _CHUNK, rbody, 0)
                pltpu.sync_copy(sva, acc.at[iab], add=True)
                pltpu.sync_copy(svb2, acc.at[ibb], add=True)

            return carry

        lax.fori_loop(0, iters, body, 0)
        plsc.subcore_barrier()
        pltpu.sync_copy(acc.at[pl.ds(sid * rz, rz)],
                        out_hbm.at[pl.ds(cid * n_pad + sid * rz, rz)])

    return _scatter


def _make_combine_body(hid, share):
    def _combine_body(pa0_ref, pa1_ref, pb0_ref, pb1_ref, fa_ref, fb_ref,
                      x_ref, out_ref):
        fa = fa_ref[0, 0]
        fb = fb_ref[0, 0]
        pa = pa0_ref[...] + pa1_ref[...]
        pb = pb0_ref[...] + pb1_ref[...]
        den = fa * pa[:, 0:hid] + fb * pb[:, 0:hid]
        num = fa * pa[:, hid:2 * hid] + fb * pb[:, hid:2 * hid]
        res = jnp.where(den > 0.0, num / den, 0.0)
        out_ref[...] = x_ref[...] + jnp.concatenate([res] * share, axis=1)
    return _combine_body


def kernel(x, x_knn, knn_idx, p_r, W, b, Wx, bx, Wp1, gamma, beta, Wp2, bp2):
    n, k, c = x_knn.shape
    hid = W.shape[1]
    share = c // hid
    m = n * k
    mh = m // 2
    f32 = jnp.float32

    xe = x_knn.reshape(m, c)
    p49 = jnp.concatenate(
        [p_r.reshape(n, k * 3), jnp.ones((n, 1), f32)], axis=1)

    q = pl.pallas_call(
        _stats_body,
        out_shape=jax.ShapeDtypeStruct((k * 3 + 1, k * 3 + 1), f32),
    )(p49)

    q48 = q[:k * 3, :k * 3].reshape(k, 3, k, 3)
    c3 = jnp.einsum('iaib->ab', q48)
    s3 = q[k * 3, :k * 3].reshape(k, 3).sum(axis=0)
    mean = (s3 / m) @ Wp1
    eh2 = jnp.einsum('ij,ik,kj->j', Wp1, c3 / m, Wp1)
    var = eh2 - mean * mean
    a = gamma * lax.rsqrt(var + 1e-5)
    cshift = beta - mean * a
    a4 = jnp.zeros((4, 4), f32).at[:3, :3].set(Wp1 * a[None, :])
    c4 = jnp.zeros((1, 4), f32).at[0, :3].set(cshift)
    b24 = jnp.zeros((4, hid), f32).at[:3, :].set(Wp2 @ W)
    b2 = (b + bp2 @ W).reshape(1, hid)
    bx2 = bx.reshape(1, hid)
    p4 = jnp.pad(p_r.reshape(m, 3), ((0, 0), (0, 1)))

    be = 640
    mA = 81920
    qA = mA // 2
    qB = (m - mA) // 2

    def run_main(o1, o2, nq):
        return pl.pallas_call(
            _main_body,
            grid=(nq,),
            in_specs=[
                pl.BlockSpec((be, c), lambda i: (i + o1, 0)),
                pl.BlockSpec((be, c), lambda i: (i + o2, 0)),
                pl.BlockSpec((be, 4), lambda i: (i + o1, 0)),
                pl.BlockSpec((be, 4), lambda i: (i + o2, 0)),
                pl.BlockSpec((c, hid), lambda i: (0, 0)),
                pl.BlockSpec((c, hid), lambda i: (0, 0)),
                pl.BlockSpec((4, 4), lambda i: (0, 0)),
                pl.BlockSpec((1, 4), lambda i: (0, 0)),
                pl.BlockSpec((4, hid), lambda i: (0, 0)),
                pl.BlockSpec((1, hid), lambda i: (0, 0)),
                pl.BlockSpec((1, hid), lambda i: (0, 0)),
            ],
            out_specs=[
                pl.BlockSpec((be, 8 * _L), lambda i: (i, 0)),
                pl.BlockSpec((1, 1), lambda i: (0, 0),
                             memory_space=pltpu.SMEM),
            ],
            out_shape=[
                jax.ShapeDtypeStruct((nq * be, 8 * _L), f32),
                jax.ShapeDtypeStruct((1, 1), f32),
            ],
        )(xe, xe, p4, p4, W, Wx, a4, c4, b24, b2, bx2)

    nqA = qA // be
    nqB = qB // be
    tpA, gA = run_main(0, nqA, nqA)
    tpB, gB = run_main(2 * nqA, 2 * nqA + nqB, nqB)

    gvA = jnp.full((_L,), gA[0, 0], f32)
    gvB = jnp.full((_L,), gB[0, 0], f32)

    n_pad = ((n + _NS * 8 - 1) // (_NS * 8)) * (_NS * 8)
    idx = knn_idx.reshape(m)
    partsA = _make_scatter(qA, n_pad)(tpA, idx[:qA], idx[qA:mA], gvA)
    partsB = _make_scatter(qB, n_pad)(
        tpB, idx[mA:mA + qB], idx[mA + qB:], gvB)
    pa0 = lax.slice(partsA, (0, 0), (n, 2 * hid))
    pa1 = lax.slice(partsA, (n_pad, 0), (n_pad + n, 2 * hid))
    pb0 = lax.slice(partsB, (0, 0), (n, 2 * hid))
    pb1 = lax.slice(partsB, (n_pad, 0), (n_pad + n, 2 * hid))

    gC = jnp.maximum(gA[0, 0], gB[0, 0])
    fa = jnp.exp(gA[0, 0] - gC).reshape(1, 1)
    fb = jnp.exp(gB[0, 0] - gC).reshape(1, 1)

    bn = 2000
    nb = n // bn
    out = pl.pallas_call(
        _make_combine_body(hid, share),
        grid=(nb,),
        in_specs=[
            pl.BlockSpec((bn, 2 * hid), lambda i: (i, 0)),
            pl.BlockSpec((bn, 2 * hid), lambda i: (i, 0)),
            pl.BlockSpec((bn, 2 * hid), lambda i: (i, 0)),
            pl.BlockSpec((bn, 2 * hid), lambda i: (i, 0)),
            pl.BlockSpec((1, 1), lambda i: (0, 0), memory_space=pltpu.SMEM),
            pl.BlockSpec((1, 1), lambda i: (0, 0), memory_space=pltpu.SMEM),
            pl.BlockSpec((bn, c), lambda i: (i, 0)),
        ],
        out_specs=pl.BlockSpec((bn, c), lambda i: (i, 0)),
        out_shape=jax.ShapeDtypeStruct((n, c), f32),
    )(pa0, pa1, pb0, pb1, fa, fb, x)
    return out


# speedup vs baseline: 1.2689x; 1.0181x over previous
import functools

import jax
import jax.numpy as jnp
from jax import lax
from jax.experimental import pallas as pl
from jax.experimental.pallas import tpu as pltpu
from jax.experimental.pallas import tpu_sc as plsc

_NC, _NS, _L = 2, 16, 16
_NW = _NC * _NS
_CHUNK = 128


def _stats_body(p_ref, q_ref):
    p = p_ref[...]
    q_ref[...] = lax.dot_general(
        p, p, (((0,), (0,)), ((), ())), preferred_element_type=jnp.float32)


def _main_body(xa_ref, xb_ref, pa_ref, pb_ref, w_ref, wx_ref, a_ref, c_ref,
               b2_ref, bias_ref, bx_ref, tp_ref, gmax_ref):
    def half(x, p):
        s = jnp.dot(x, w_ref[...], preferred_element_type=jnp.float32)
        hr = jnp.maximum(
            jnp.dot(p, a_ref[...],
                    preferred_element_type=jnp.float32) + c_ref[...], 0.0)
        s = s + jnp.dot(hr, b2_ref[...],
                        preferred_element_type=jnp.float32) + bias_ref[...]
        v = jnp.dot(x, wx_ref[...], preferred_element_type=jnp.float32) + bx_ref[...]
        return s, v

    sa, va = half(xa_ref[...], pa_ref[...])
    sb, vb = half(xb_ref[...], pb_ref[...])
    tp_ref[...] = jnp.concatenate([sa, va, sb, vb], axis=1)

    @pl.when(pl.program_id(0) == 0)
    def _():
        gmax_ref[0, 0] = -jnp.inf

    gmax_ref[0, 0] = jnp.maximum(
        gmax_ref[0, 0], jnp.maximum(jnp.max(sa), jnp.max(sb)))


def _make_scatter(mh, n_pad):
    nchunks = mh // _CHUNK
    iters = (nchunks + _NW - 1) // _NW
    rz = n_pad // _NS
    mesh = plsc.VectorSubcoreMesh(core_axis_name="c", subcore_axis_name="s")

    @functools.partial(
        pl.kernel,
        out_type=[
            jax.ShapeDtypeStruct((n_pad, 8 * _L), jnp.float32),
            jax.ShapeDtypeStruct((n_pad, 8 * _L), jnp.float32),
        ],
        mesh=mesh,
        scratch_types=[
            pltpu.VMEM((_CHUNK,), jnp.int32),
            pltpu.VMEM((_CHUNK,), jnp.int32),
            pltpu.VMEM((_CHUNK, 8 * _L), jnp.float32),
            pltpu.VMEM((_CHUNK, 8 * _L), jnp.float32),
            pltpu.VMEM((_CHUNK, 8 * _L), jnp.float32),
            pltpu.VMEM((_L,), jnp.float32),
            pltpu.VMEM_SHARED((n_pad, 8 * _L), jnp.float32),
        ],
    )
    def _scatter(tp_hbm, ia_hbm, ib_hbm, g_hbm, out0_hbm, out1_hbm,
                 iab, ibb, svb, sva, svb2, gb, acc):
        cid = lax.axis_index("c")
        sid = lax.axis_index("s")
        wid = sid * _NC + cid

        def zpay(r, carry):
            for j in range(8):
                svb[r, pl.ds(j * _L, _L)] = jnp.zeros((_L,), jnp.float32)
            for j in range(4, 8):
                sva[r, pl.ds(j * _L, _L)] = jnp.zeros((_L,), jnp.float32)
                svb2[r, pl.ds(j * _L, _L)] = jnp.zeros((_L,), jnp.float32)
            return carry

        lax.fori_loop(0, _CHUNK, zpay, 0)

        def zslab(r, carry):
            pltpu.sync_copy(svb.at[pl.ds(0, 8)],
                            acc.at[pl.ds(sid * rz + r * 8, 8)])
            return carry

        lax.fori_loop(0, rz // 8, zslab, 0)
        pltpu.sync_copy(g_hbm, gb)
        gv = gb[...]
        plsc.subcore_barrier()

        def body(j, carry):
            chunk = wid + _NW * j

            @pl.when(chunk < nchunks)
            def _():
                base = chunk * _CHUNK
                pltpu.sync_copy(ia_hbm.at[pl.ds(base, _CHUNK)], iab)
                pltpu.sync_copy(ib_hbm.at[pl.ds(base, _CHUNK)], ibb)
                pltpu.sync_copy(tp_hbm.at[pl.ds(base, _CHUNK)], svb)

                def rbody(r, c2):
                    ea0 = jnp.exp(svb[r, pl.ds(0, _L)] - gv)
                    ea1 = jnp.exp(svb[r, pl.ds(_L, _L)] - gv)
                    eb0 = jnp.exp(svb[r, pl.ds(4 * _L, _L)] - gv)
                    eb1 = jnp.exp(svb[r, pl.ds(5 * _L, _L)] - gv)
                    sva[r, pl.ds(0, _L)] = ea0
                    sva[r, pl.ds(_L, _L)] = ea1
                    sva[r, pl.ds(2 * _L, _L)] = svb[r, pl.ds(2 * _L, _L)] * ea0
                    sva[r, pl.ds(3 * _L, _L)] = svb[r, pl.ds(3 * _L, _L)] * ea1
                    svb2[r, pl.ds(0, _L)] = eb0
                    svb2[r, pl.ds(_L, _L)] = eb1
                    svb2[r, pl.ds(2 * _L, _L)] = svb[r, pl.ds(6 * _L, _L)] * eb0
                    svb2[r, pl.ds(3 * _L, _L)] = svb[r, pl.ds(7 * _L, _L)] * eb1
                    return c2

                lax.fori_loop(0, _CHUNK, rbody, 0)
                pltpu.sync_copy(sva, acc.at[iab], add=True)
                pltpu.sync_copy(svb2, acc.at[ibb], add=True)

            return carry

        lax.fori_loop(0, iters, body, 0)
        plsc.subcore_barrier()

        @pl.when(cid == 0)
        def _():
            pltpu.sync_copy(acc.at[pl.ds(sid * rz, rz)],
                            out0_hbm.at[pl.ds(sid * rz, rz)])

        @pl.when(cid == 1)
        def _():
            pltpu.sync_copy(acc.at[pl.ds(sid * rz, rz)],
                            out1_hbm.at[pl.ds(sid * rz, rz)])

    return _scatter


def _make_combine_body(hid, share):
    def _combine_body(pa0_ref, pa1_ref, pb0_ref, pb1_ref, fa_ref, fb_ref,
                      x_ref, out_ref):
        fa = fa_ref[0, 0]
        fb = fb_ref[0, 0]
        pa = pa0_ref[...] + pa1_ref[...]
        pb = pb0_ref[...] + pb1_ref[...]
        den = fa * pa[:, 0:hid] + fb * pb[:, 0:hid]
        num = fa * pa[:, hid:2 * hid] + fb * pb[:, hid:2 * hid]
        res = jnp.where(den > 0.0, num / den, 0.0)
        out_ref[...] = x_ref[...] + jnp.concatenate([res] * share, axis=1)
    return _combine_body


def kernel(x, x_knn, knn_idx, p_r, W, b, Wx, bx, Wp1, gamma, beta, Wp2, bp2):
    n, k, c = x_knn.shape
    hid = W.shape[1]
    share = c // hid
    m = n * k
    mh = m // 2
    f32 = jnp.float32

    xe = x_knn.reshape(m, c)
    p49 = jnp.concatenate(
        [p_r.reshape(n, k * 3), jnp.ones((n, 1), f32)], axis=1)

    q = pl.pallas_call(
        _stats_body,
        out_shape=jax.ShapeDtypeStruct((k * 3 + 1, k * 3 + 1), f32),
    )(p49)

    q48 = q[:k * 3, :k * 3].reshape(k, 3, k, 3)
    c3 = jnp.einsum('iaib->ab', q48)
    s3 = q[k * 3, :k * 3].reshape(k, 3).sum(axis=0)
    mean = (s3 / m) @ Wp1
    eh2 = jnp.einsum('ij,ik,kj->j', Wp1, c3 / m, Wp1)
    var = eh2 - mean * mean
    a = gamma * lax.rsqrt(var + 1e-5)
    cshift = beta - mean * a
    a4 = jnp.zeros((4, 4), f32).at[:3, :3].set(Wp1 * a[None, :])
    c4 = jnp.zeros((1, 4), f32).at[0, :3].set(cshift)
    b24 = jnp.zeros((4, hid), f32).at[:3, :].set(Wp2 @ W)
    b2 = (b + bp2 @ W).reshape(1, hid)
    bx2 = bx.reshape(1, hid)
    p4 = jnp.pad(p_r.reshape(m, 3), ((0, 0), (0, 1)))

    be = 640
    mA = 81920
    qA = mA // 2
    qB = (m - mA) // 2

    def run_main(o1, o2, nq):
        return pl.pallas_call(
            _main_body,
            grid=(nq,),
            in_specs=[
                pl.BlockSpec((be, c), lambda i: (i + o1, 0)),
                pl.BlockSpec((be, c), lambda i: (i + o2, 0)),
                pl.BlockSpec((be, 4), lambda i: (i + o1, 0)),
                pl.BlockSpec((be, 4), lambda i: (i + o2, 0)),
                pl.BlockSpec((c, hid), lambda i: (0, 0)),
                pl.BlockSpec((c, hid), lambda i: (0, 0)),
                pl.BlockSpec((4, 4), lambda i: (0, 0)),
                pl.BlockSpec((1, 4), lambda i: (0, 0)),
                pl.BlockSpec((4, hid), lambda i: (0, 0)),
                pl.BlockSpec((1, hid), lambda i: (0, 0)),
                pl.BlockSpec((1, hid), lambda i: (0, 0)),
            ],
            out_specs=[
                pl.BlockSpec((be, 8 * _L), lambda i: (i, 0)),
                pl.BlockSpec((1, 1), lambda i: (0, 0),
                             memory_space=pltpu.SMEM),
            ],
            out_shape=[
                jax.ShapeDtypeStruct((nq * be, 8 * _L), f32),
                jax.ShapeDtypeStruct((1, 1), f32),
            ],
        )(xe, xe, p4, p4, W, Wx, a4, c4, b24, b2, bx2)

    nqA = qA // be
    nqB = qB // be
    tpA, gA = run_main(0, nqA, nqA)
    tpB, gB = run_main(2 * nqA, 2 * nqA + nqB, nqB)

    gvA = jnp.full((_L,), gA[0, 0], f32)
    gvB = jnp.full((_L,), gB[0, 0], f32)

    n_pad = ((n + _NS * 8 - 1) // (_NS * 8)) * (_NS * 8)
    idx = knn_idx.reshape(m)
    pa0, pa1 = _make_scatter(qA, n_pad)(tpA, idx[:qA], idx[qA:mA], gvA)
    pb0, pb1 = _make_scatter(qB, n_pad)(
        tpB, idx[mA:mA + qB], idx[mA + qB:], gvB)

    gC = jnp.maximum(gA[0, 0], gB[0, 0])
    fa = jnp.exp(gA[0, 0] - gC).reshape(1, 1)
    fb = jnp.exp(gB[0, 0] - gC).reshape(1, 1)

    bn = 2000
    nb = n // bn
    out = pl.pallas_call(
        _make_combine_body(hid, share),
        grid=(nb,),
        in_specs=[
            pl.BlockSpec((bn, 8 * _L), lambda i: (i, 0)),
            pl.BlockSpec((bn, 8 * _L), lambda i: (i, 0)),
            pl.BlockSpec((bn, 8 * _L), lambda i: (i, 0)),
            pl.BlockSpec((bn, 8 * _L), lambda i: (i, 0)),
            pl.BlockSpec((1, 1), lambda i: (0, 0), memory_space=pltpu.SMEM),
            pl.BlockSpec((1, 1), lambda i: (0, 0), memory_space=pltpu.SMEM),
            pl.BlockSpec((bn, c), lambda i: (i, 0)),
        ],
        out_specs=pl.BlockSpec((bn, c), lambda i: (i, 0)),
        out_shape=jax.ShapeDtypeStruct((n, c), f32),
    )(pa0, pa1, pb0, pb1, fa, fb, x)
    return out


# speedup vs baseline: 1.2933x; 1.0192x over previous
import functools

import jax
import jax.numpy as jnp
from jax import lax
from jax.experimental import pallas as pl
from jax.experimental.pallas import tpu as pltpu
from jax.experimental.pallas import tpu_sc as plsc

_NC, _NS, _L = 2, 16, 16
_NW = _NC * _NS
_CHUNK = 80


def _stats_body(p_ref, q_ref):
    p = p_ref[...]
    q_ref[...] = lax.dot_general(
        p, p, (((0,), (0,)), ((), ())), preferred_element_type=jnp.float32)


def _main_body(xa_ref, xb_ref, pa_ref, pb_ref, w_ref, wx_ref, a_ref, c_ref,
               b2_ref, bias_ref, bx_ref, tp_ref, gmax_ref):
    def half(x, p):
        s = jnp.dot(x, w_ref[...], preferred_element_type=jnp.float32)
        hr = jnp.maximum(
            jnp.dot(p, a_ref[...],
                    preferred_element_type=jnp.float32) + c_ref[...], 0.0)
        s = s + jnp.dot(hr, b2_ref[...],
                        preferred_element_type=jnp.float32) + bias_ref[...]
        v = jnp.dot(x, wx_ref[...], preferred_element_type=jnp.float32) + bx_ref[...]
        return s, v

    sa, va = half(xa_ref[...], pa_ref[...])
    sb, vb = half(xb_ref[...], pb_ref[...])
    tp_ref[...] = jnp.concatenate([sa, va, sb, vb], axis=1)

    @pl.when(pl.program_id(0) == 0)
    def _():
        gmax_ref[0, 0] = -jnp.inf

    gmax_ref[0, 0] = jnp.maximum(
        gmax_ref[0, 0], jnp.maximum(jnp.max(sa), jnp.max(sb)))


def _make_scatter(mh, n_pad):
    nchunks = mh // _CHUNK
    iters = (nchunks + _NW - 1) // _NW
    rz = n_pad // _NS
    mesh = plsc.VectorSubcoreMesh(core_axis_name="c", subcore_axis_name="s")

    @functools.partial(
        pl.kernel,
        out_type=[
            jax.ShapeDtypeStruct((n_pad, 8 * _L), jnp.float32),
            jax.ShapeDtypeStruct((n_pad, 8 * _L), jnp.float32),
        ],
        mesh=mesh,
        scratch_types=[
            pltpu.VMEM((_CHUNK,), jnp.int32),
            pltpu.VMEM((_CHUNK,), jnp.int32),
            pltpu.VMEM((_CHUNK, 8 * _L), jnp.float32),
            pltpu.VMEM((_CHUNK, 8 * _L), jnp.float32),
            pltpu.VMEM((_CHUNK, 8 * _L), jnp.float32),
            pltpu.VMEM((_CHUNK, 8 * _L), jnp.float32),
            pltpu.VMEM((_L,), jnp.float32),
            pltpu.SemaphoreType.DMA,
            pltpu.SemaphoreType.DMA,
            pltpu.VMEM_SHARED((n_pad, 8 * _L), jnp.float32),
        ],
    )
    def _scatter(tp_hbm, ia_hbm, ib_hbm, g_hbm, out0_hbm, out1_hbm,
                 iab, ibb, svb0, svb1, sva, svb2, gb, sem0, sem1, acc):
        cid = lax.axis_index("c")
        sid = lax.axis_index("s")
        wid = sid * _NC + cid

        def zpay(r, carry):
            for j in range(8):
                svb0[r, pl.ds(j * _L, _L)] = jnp.zeros((_L,), jnp.float32)
            for j in range(4, 8):
                sva[r, pl.ds(j * _L, _L)] = jnp.zeros((_L,), jnp.float32)
                svb2[r, pl.ds(j * _L, _L)] = jnp.zeros((_L,), jnp.float32)
            return carry

        lax.fori_loop(0, _CHUNK, zpay, 0)

        def zslab(r, carry):
            pltpu.sync_copy(svb0.at[pl.ds(0, 8)],
                            acc.at[pl.ds(sid * rz + r * 8, 8)])
            return carry

        lax.fori_loop(0, rz // 8, zslab, 0)
        pltpu.sync_copy(g_hbm, gb)
        gv = gb[...]
        plsc.subcore_barrier()

        def start_load(j, buf, sem):
            chunk = wid + _NW * j

            @pl.when(chunk < nchunks)
            def _():
                pltpu.async_copy(
                    tp_hbm.at[pl.ds(chunk * _CHUNK, _CHUNK)], buf, sem)

        def process(j, buf, sem):
            chunk = wid + _NW * j

            @pl.when(chunk < nchunks)
            def _():
                base = chunk * _CHUNK
                pltpu.sync_copy(ia_hbm.at[pl.ds(base, _CHUNK)], iab)
                pltpu.sync_copy(ib_hbm.at[pl.ds(base, _CHUNK)], ibb)
                pltpu.make_async_copy(
                    tp_hbm.at[pl.ds(base, _CHUNK)], buf, sem).wait()

                def rbody(r, c2):
                    ea0 = jnp.exp(buf[r, pl.ds(0, _L)] - gv)
                    ea1 = jnp.exp(buf[r, pl.ds(_L, _L)] - gv)
                    eb0 = jnp.exp(buf[r, pl.ds(4 * _L, _L)] - gv)
                    eb1 = jnp.exp(buf[r, pl.ds(5 * _L, _L)] - gv)
                    sva[r, pl.ds(0, _L)] = ea0
                    sva[r, pl.ds(_L, _L)] = ea1
                    sva[r, pl.ds(2 * _L, _L)] = buf[r, pl.ds(2 * _L, _L)] * ea0
                    sva[r, pl.ds(3 * _L, _L)] = buf[r, pl.ds(3 * _L, _L)] * ea1
                    svb2[r, pl.ds(0, _L)] = eb0
                    svb2[r, pl.ds(_L, _L)] = eb1
                    svb2[r, pl.ds(2 * _L, _L)] = buf[r, pl.ds(6 * _L, _L)] * eb0
                    svb2[r, pl.ds(3 * _L, _L)] = buf[r, pl.ds(7 * _L, _L)] * eb1
                    return c2

                lax.fori_loop(0, _CHUNK, rbody, 0)
                pltpu.sync_copy(sva, acc.at[iab], add=True)
                pltpu.sync_copy(svb2, acc.at[ibb], add=True)

        start_load(0, svb0, sem0)

        def pair(g, carry):
            j0 = 2 * g
            start_load(j0 + 1, svb1, sem1)
            process(j0, svb0, sem0)
            start_load(j0 + 2, svb0, sem0)
            process(j0 + 1, svb1, sem1)
            return carry

        lax.fori_loop(0, (iters + 1) // 2, pair, 0)
        plsc.subcore_barrier()

        @pl.when(cid == 0)
        def _():
            pltpu.sync_copy(acc.at[pl.ds(sid * rz, rz)],
                            out0_hbm.at[pl.ds(sid * rz, rz)])

        @pl.when(cid == 1)
        def _():
            pltpu.sync_copy(acc.at[pl.ds(sid * rz, rz)],
                            out1_hbm.at[pl.ds(sid * rz, rz)])

    return _scatter


def _make_combine_body(hid, share):
    def _combine_body(pa0_ref, pa1_ref, pb0_ref, pb1_ref, fa_ref, fb_ref,
                      x_ref, out_ref):
        fa = fa_ref[0, 0]
        fb = fb_ref[0, 0]
        pa = pa0_ref[...] + pa1_ref[...]
        pb = pb0_ref[...] + pb1_ref[...]
        den = fa * pa[:, 0:hid] + fb * pb[:, 0:hid]
        num = fa * pa[:, hid:2 * hid] + fb * pb[:, hid:2 * hid]
        res = jnp.where(den > 0.0, num / den, 0.0)
        out_ref[...] = x_ref[...] + jnp.concatenate([res] * share, axis=1)
    return _combine_body


def kernel(x, x_knn, knn_idx, p_r, W, b, Wx, bx, Wp1, gamma, beta, Wp2, bp2):
    n, k, c = x_knn.shape
    hid = W.shape[1]
    share = c // hid
    m = n * k
    mh = m // 2
    f32 = jnp.float32

    xe = x_knn.reshape(m, c)
    p49 = jnp.concatenate(
        [p_r.reshape(n, k * 3), jnp.ones((n, 1), f32)], axis=1)

    q = pl.pallas_call(
        _stats_body,
        out_shape=jax.ShapeDtypeStruct((k * 3 + 1, k * 3 + 1), f32),
    )(p49)

    q48 = q[:k * 3, :k * 3].reshape(k, 3, k, 3)
    c3 = jnp.einsum('iaib->ab', q48)
    s3 = q[k * 3, :k * 3].reshape(k, 3).sum(axis=0)
    mean = (s3 / m) @ Wp1
    eh2 = jnp.einsum('ij,ik,kj->j', Wp1, c3 / m, Wp1)
    var = eh2 - mean * mean
    a = gamma * lax.rsqrt(var + 1e-5)
    cshift = beta - mean * a
    a4 = jnp.zeros((4, 4), f32).at[:3, :3].set(Wp1 * a[None, :])
    c4 = jnp.zeros((1, 4), f32).at[0, :3].set(cshift)
    b24 = jnp.zeros((4, hid), f32).at[:3, :].set(Wp2 @ W)
    b2 = (b + bp2 @ W).reshape(1, hid)
    bx2 = bx.reshape(1, hid)
    p4 = jnp.pad(p_r.reshape(m, 3), ((0, 0), (0, 1)))

    be = 640
    mA = 81920
    qA = mA // 2
    qB = (m - mA) // 2

    def run_main(o1, o2, nq):
        return pl.pallas_call(
            _main_body,
            grid=(nq,),
            in_specs=[
                pl.BlockSpec((be, c), lambda i: (i + o1, 0)),
                pl.BlockSpec((be, c), lambda i: (i + o2, 0)),
                pl.BlockSpec((be, 4), lambda i: (i + o1, 0)),
                pl.BlockSpec((be, 4), lambda i: (i + o2, 0)),
                pl.BlockSpec((c, hid), lambda i: (0, 0)),
                pl.BlockSpec((c, hid), lambda i: (0, 0)),
                pl.BlockSpec((4, 4), lambda i: (0, 0)),
                pl.BlockSpec((1, 4), lambda i: (0, 0)),
                pl.BlockSpec((4, hid), lambda i: (0, 0)),
                pl.BlockSpec((1, hid), lambda i: (0, 0)),
                pl.BlockSpec((1, hid), lambda i: (0, 0)),
            ],
            out_specs=[
                pl.BlockSpec((be, 8 * _L), lambda i: (i, 0)),
                pl.BlockSpec((1, 1), lambda i: (0, 0),
                             memory_space=pltpu.SMEM),
            ],
            out_shape=[
                jax.ShapeDtypeStruct((nq * be, 8 * _L), f32),
                jax.ShapeDtypeStruct((1, 1), f32),
            ],
        )(xe, xe, p4, p4, W, Wx, a4, c4, b24, b2, bx2)

    nqA = qA // be
    nqB = qB // be
    tpA, gA = run_main(0, nqA, nqA)
    tpB, gB = run_main(2 * nqA, 2 * nqA + nqB, nqB)

    gvA = jnp.full((_L,), gA[0, 0], f32)
    gvB = jnp.full((_L,), gB[0, 0], f32)

    n_pad = ((n + _NS * 8 - 1) // (_NS * 8)) * (_NS * 8)
    idx = knn_idx.reshape(m)
    pa0, pa1 = _make_scatter(qA, n_pad)(tpA, idx[:qA], idx[qA:mA], gvA)
    pb0, pb1 = _make_scatter(qB, n_pad)(
        tpB, idx[mA:mA + qB], idx[mA + qB:], gvB)

    gC = jnp.maximum(gA[0, 0], gB[0, 0])
    fa = jnp.exp(gA[0, 0] - gC).reshape(1, 1)
    fb = jnp.exp(gB[0, 0] - gC).reshape(1, 1)

    bn = 2000
    nb = n // bn
    out = pl.pallas_call(
        _make_combine_body(hid, share),
        grid=(nb,),
        in_specs=[
            pl.BlockSpec((bn, 8 * _L), lambda i: (i, 0)),
            pl.BlockSpec((bn, 8 * _L), lambda i: (i, 0)),
            pl.BlockSpec((bn, 8 * _L), lambda i: (i, 0)),
            pl.BlockSpec((bn, 8 * _L), lambda i: (i, 0)),
            pl.BlockSpec((1, 1), lambda i: (0, 0), memory_space=pltpu.SMEM),
            pl.BlockSpec((1, 1), lambda i: (0, 0), memory_space=pltpu.SMEM),
            pl.BlockSpec((bn, c), lambda i: (i, 0)),
        ],
        out_specs=pl.BlockSpec((bn, c), lambda i: (i, 0)),
        out_shape=jax.ShapeDtypeStruct((n, c), f32),
    )(pa0, pa1, pb0, pb1, fa, fb, x)
    return out


# speedup vs baseline: 1.3350x; 1.0322x over previous
import functools

import jax
import jax.numpy as jnp
from jax import lax
from jax.experimental import pallas as pl
from jax.experimental.pallas import tpu as pltpu
from jax.experimental.pallas import tpu_sc as plsc

_NC, _NS, _L = 2, 16, 16
_NW = _NC * _NS
_CHUNK = 80


def _stats_body(p_ref, q_ref):
    p = p_ref[...]
    q_ref[...] = lax.dot_general(
        p, p, (((0,), (0,)), ((), ())), preferred_element_type=jnp.float32)


def _main_body(xa_ref, xb_ref, pa_ref, pb_ref, w_ref, wx_ref, a_ref, c_ref,
               b2_ref, bias_ref, bx_ref, tp_ref, gmax_ref):
    def half(x, p):
        s = jnp.dot(x, w_ref[...], preferred_element_type=jnp.float32)
        hr = jnp.maximum(
            jnp.dot(p, a_ref[...],
                    preferred_element_type=jnp.float32) + c_ref[...], 0.0)
        s = s + jnp.dot(hr, b2_ref[...],
                        preferred_element_type=jnp.float32) + bias_ref[...]
        v = jnp.dot(x, wx_ref[...], preferred_element_type=jnp.float32) + bx_ref[...]
        return s, v

    sa, va = half(xa_ref[...], pa_ref[...])
    sb, vb = half(xb_ref[...], pb_ref[...])
    tp_ref[...] = jnp.concatenate([sa, va, sb, vb], axis=1)

    @pl.when(pl.program_id(0) == 0)
    def _():
        gmax_ref[0, 0] = -jnp.inf

    gmax_ref[0, 0] = jnp.maximum(
        gmax_ref[0, 0], jnp.maximum(jnp.max(sa), jnp.max(sb)))


def _make_scatter(mh, n_pad):
    nchunks = mh // _CHUNK
    iters = (nchunks + _NW - 1) // _NW
    rz = n_pad // _NS
    mesh = plsc.VectorSubcoreMesh(core_axis_name="c", subcore_axis_name="s")

    @functools.partial(
        pl.kernel,
        out_type=[
            jax.ShapeDtypeStruct((n_pad, 8 * _L), jnp.float32),
            jax.ShapeDtypeStruct((n_pad, 8 * _L), jnp.float32),
        ],
        mesh=mesh,
        scratch_types=[
            pltpu.VMEM((_CHUNK,), jnp.int32),
            pltpu.VMEM((_CHUNK,), jnp.int32),
            pltpu.VMEM((_CHUNK,), jnp.int32),
            pltpu.VMEM((_CHUNK,), jnp.int32),
            pltpu.VMEM((_CHUNK, 8 * _L), jnp.float32),
            pltpu.VMEM((_CHUNK, 8 * _L), jnp.float32),
            pltpu.VMEM((_CHUNK, 8 * _L), jnp.float32),
            pltpu.VMEM((_CHUNK, 8 * _L), jnp.float32),
            pltpu.VMEM((_L,), jnp.float32),
            pltpu.SemaphoreType.DMA,
            pltpu.SemaphoreType.DMA,
            pltpu.VMEM_SHARED((n_pad, 8 * _L), jnp.float32),
        ],
    )
    def _scatter(tp_hbm, ia_hbm, ib_hbm, g_hbm, out0_hbm, out1_hbm,
                 iab0, iab1, ibb0, ibb1, svb0, svb1, sva, svb2, gb,
                 sem0, sem1, acc):
        cid = lax.axis_index("c")
        sid = lax.axis_index("s")
        wid = sid * _NC + cid

        def zpay(r, carry):
            for j in range(8):
                svb0[r, pl.ds(j * _L, _L)] = jnp.zeros((_L,), jnp.float32)
            for j in range(4, 8):
                sva[r, pl.ds(j * _L, _L)] = jnp.zeros((_L,), jnp.float32)
                svb2[r, pl.ds(j * _L, _L)] = jnp.zeros((_L,), jnp.float32)
            return carry

        lax.fori_loop(0, _CHUNK, zpay, 0)

        def zslab(r, carry):
            pltpu.sync_copy(svb0.at[pl.ds(0, 8)],
                            acc.at[pl.ds(sid * rz + r * 8, 8)])
            return carry

        lax.fori_loop(0, rz // 8, zslab, 0)
        pltpu.sync_copy(g_hbm, gb)
        gv = gb[...]
        plsc.subcore_barrier()

        def start_load(j, buf, ia, ib, sem):
            chunk = wid + _NW * j

            @pl.when(chunk < nchunks)
            def _():
                base = chunk * _CHUNK
                pltpu.async_copy(ia_hbm.at[pl.ds(base, _CHUNK)], ia, sem)
                pltpu.async_copy(ib_hbm.at[pl.ds(base, _CHUNK)], ib, sem)
                pltpu.async_copy(tp_hbm.at[pl.ds(base, _CHUNK)], buf, sem)

        def process(j, buf, ia, ib, sem):
            chunk = wid + _NW * j

            @pl.when(chunk < nchunks)
            def _():
                base = chunk * _CHUNK
                pltpu.make_async_copy(
                    ia_hbm.at[pl.ds(base, _CHUNK)], ia, sem).wait()
                pltpu.make_async_copy(
                    ib_hbm.at[pl.ds(base, _CHUNK)], ib, sem).wait()
                pltpu.make_async_copy(
                    tp_hbm.at[pl.ds(base, _CHUNK)], buf, sem).wait()

                def rbody(r, c2):
                    ea0 = jnp.exp(buf[r, pl.ds(0, _L)] - gv)
                    ea1 = jnp.exp(buf[r, pl.ds(_L, _L)] - gv)
                    eb0 = jnp.exp(buf[r, pl.ds(4 * _L, _L)] - gv)
                    eb1 = jnp.exp(buf[r, pl.ds(5 * _L, _L)] - gv)
                    sva[r, pl.ds(0, _L)] = ea0
                    sva[r, pl.ds(_L, _L)] = ea1
                    sva[r, pl.ds(2 * _L, _L)] = buf[r, pl.ds(2 * _L, _L)] * ea0
                    sva[r, pl.ds(3 * _L, _L)] = buf[r, pl.ds(3 * _L, _L)] * ea1
                    svb2[r, pl.ds(0, _L)] = eb0
                    svb2[r, pl.ds(_L, _L)] = eb1
                    svb2[r, pl.ds(2 * _L, _L)] = buf[r, pl.ds(6 * _L, _L)] * eb0
                    svb2[r, pl.ds(3 * _L, _L)] = buf[r, pl.ds(7 * _L, _L)] * eb1
                    return c2

                lax.fori_loop(0, _CHUNK, rbody, 0)
                pltpu.sync_copy(sva, acc.at[ia], add=True)
                pltpu.sync_copy(svb2, acc.at[ib], add=True)

        start_load(0, svb0, iab0, ibb0, sem0)

        def pair(g, carry):
            j0 = 2 * g
            start_load(j0 + 1, svb1, iab1, ibb1, sem1)
            process(j0, svb0, iab0, ibb0, sem0)
            start_load(j0 + 2, svb0, iab0, ibb0, sem0)
            process(j0 + 1, svb1, iab1, ibb1, sem1)
            return carry

        lax.fori_loop(0, (iters + 1) // 2, pair, 0)
        plsc.subcore_barrier()

        @pl.when(cid == 0)
        def _():
            pltpu.sync_copy(acc.at[pl.ds(sid * rz, rz)],
                            out0_hbm.at[pl.ds(sid * rz, rz)])

        @pl.when(cid == 1)
        def _():
            pltpu.sync_copy(acc.at[pl.ds(sid * rz, rz)],
                            out1_hbm.at[pl.ds(sid * rz, rz)])

    return _scatter


def _make_combine_body(hid, share):
    def _combine_body(pa0_ref, pa1_ref, pb0_ref, pb1_ref, fa_ref, fb_ref,
                      x_ref, out_ref):
        fa = fa_ref[0, 0]
        fb = fb_ref[0, 0]
        pa = pa0_ref[...] + pa1_ref[...]
        pb = pb0_ref[...] + pb1_ref[...]
        den = fa * pa[:, 0:hid] + fb * pb[:, 0:hid]
        num = fa * pa[:, hid:2 * hid] + fb * pb[:, hid:2 * hid]
        res = jnp.where(den > 0.0, num / den, 0.0)
        out_ref[...] = x_ref[...] + jnp.concatenate([res] * share, axis=1)
    return _combine_body


def kernel(x, x_knn, knn_idx, p_r, W, b, Wx, bx, Wp1, gamma, beta, Wp2, bp2):
    n, k, c = x_knn.shape
    hid = W.shape[1]
    share = c // hid
    m = n * k
    mh = m // 2
    f32 = jnp.float32

    xe = x_knn.reshape(m, c)
    p49 = jnp.concatenate(
        [p_r.reshape(n, k * 3), jnp.ones((n, 1), f32)], axis=1)

    q = pl.pallas_call(
        _stats_body,
        out_shape=jax.ShapeDtypeStruct((k * 3 + 1, k * 3 + 1), f32),
    )(p49)

    q48 = q[:k * 3, :k * 3].reshape(k, 3, k, 3)
    c3 = jnp.einsum('iaib->ab', q48)
    s3 = q[k * 3, :k * 3].reshape(k, 3).sum(axis=0)
    mean = (s3 / m) @ Wp1
    eh2 = jnp.einsum('ij,ik,kj->j', Wp1, c3 / m, Wp1)
    var = eh2 - mean * mean
    a = gamma * lax.rsqrt(var + 1e-5)
    cshift = beta - mean * a
    a4 = jnp.zeros((4, 4), f32).at[:3, :3].set(Wp1 * a[None, :])
    c4 = jnp.zeros((1, 4), f32).at[0, :3].set(cshift)
    b24 = jnp.zeros((4, hid), f32).at[:3, :].set(Wp2 @ W)
    b2 = (b + bp2 @ W).reshape(1, hid)
    bx2 = bx.reshape(1, hid)
    p4 = jnp.pad(p_r.reshape(m, 3), ((0, 0), (0, 1)))

    be = 640
    mA = 81920
    qA = mA // 2
    qB = (m - mA) // 2

    def run_main(o1, o2, nq):
        return pl.pallas_call(
            _main_body,
            grid=(nq,),
            in_specs=[
                pl.BlockSpec((be, c), lambda i: (i + o1, 0)),
                pl.BlockSpec((be, c), lambda i: (i + o2, 0)),
                pl.BlockSpec((be, 4), lambda i: (i + o1, 0)),
                pl.BlockSpec((be, 4), lambda i: (i + o2, 0)),
                pl.BlockSpec((c, hid), lambda i: (0, 0)),
                pl.BlockSpec((c, hid), lambda i: (0, 0)),
                pl.BlockSpec((4, 4), lambda i: (0, 0)),
                pl.BlockSpec((1, 4), lambda i: (0, 0)),
                pl.BlockSpec((4, hid), lambda i: (0, 0)),
                pl.BlockSpec((1, hid), lambda i: (0, 0)),
                pl.BlockSpec((1, hid), lambda i: (0, 0)),
            ],
            out_specs=[
                pl.BlockSpec((be, 8 * _L), lambda i: (i, 0)),
                pl.BlockSpec((1, 1), lambda i: (0, 0),
                             memory_space=pltpu.SMEM),
            ],
            out_shape=[
                jax.ShapeDtypeStruct((nq * be, 8 * _L), f32),
                jax.ShapeDtypeStruct((1, 1), f32),
            ],
        )(xe, xe, p4, p4, W, Wx, a4, c4, b24, b2, bx2)

    nqA = qA // be
    nqB = qB // be
    tpA, gA = run_main(0, nqA, nqA)
    tpB, gB = run_main(2 * nqA, 2 * nqA + nqB, nqB)

    gvA = jnp.full((_L,), gA[0, 0], f32)
    gvB = jnp.full((_L,), gB[0, 0], f32)

    n_pad = ((n + _NS * 8 - 1) // (_NS * 8)) * (_NS * 8)
    idx = knn_idx.reshape(m)
    pa0, pa1 = _make_scatter(qA, n_pad)(tpA, idx[:qA], idx[qA:mA], gvA)
    pb0, pb1 = _make_scatter(qB, n_pad)(
        tpB, idx[mA:mA + qB], idx[mA + qB:], gvB)

    gC = jnp.maximum(gA[0, 0], gB[0, 0])
    fa = jnp.exp(gA[0, 0] - gC).reshape(1, 1)
    fb = jnp.exp(gB[0, 0] - gC).reshape(1, 1)

    bn = 2000
    nb = n // bn
    out = pl.pallas_call(
        _make_combine_body(hid, share),
        grid=(nb,),
        in_specs=[
            pl.BlockSpec((bn, 8 * _L), lambda i: (i, 0)),
            pl.BlockSpec((bn, 8 * _L), lambda i: (i, 0)),
            pl.BlockSpec((bn, 8 * _L), lambda i: (i, 0)),
            pl.BlockSpec((bn, 8 * _L), lambda i: (i, 0)),
            pl.BlockSpec((1, 1), lambda i: (0, 0), memory_space=pltpu.SMEM),
            pl.BlockSpec((1, 1), lambda i: (0, 0), memory_space=pltpu.SMEM),
            pl.BlockSpec((bn, c), lambda i: (i, 0)),
        ],
        out_specs=pl.BlockSpec((bn, c), lambda i: (i, 0)),
        out_shape=jax.ShapeDtypeStruct((n, c), f32),
    )(pa0, pa1, pb0, pb1, fa, fb, x)
    return out
